# baseline (speedup 1.0000x reference)
import functools

import jax
import jax.numpy as jnp
from jax import lax
from jax.experimental import pallas as pl
from jax.experimental.pallas import tpu as pltpu


def _sig(v):
    return 0.5 * jnp.tanh(0.5 * v) + 0.5


def _dot_tb(a, b_raw):
    return lax.dot_general(a, b_raw, (((1,), (1,)), ((), ())),
                           preferred_element_type=jnp.float32)


def _lstm_kernel(x_ref, wih_ref, whh_ref, b_ref, wfc_ref, bfc_ref, out_ref,
                 h_sc, c_sc, xt_sc, pre_sc, whht_sc, sem,
                 *, tc, bp, h_dim, grp, nc):
    k = pl.program_id(0)
    bf16 = jnp.bfloat16
    n_grp = bp // grp
    buf = lax.rem(k, 2)
    prev = lax.rem(k + 1, 2)

    @pl.when(k == 0)
    def _weights():
        whht_sc[...] = jnp.transpose(whh_ref[...]).astype(bf16)

    @pl.when(k >= 1)
    def _wait():
        for tl in range(tc):
            pltpu.make_async_copy(
                x_ref.at[:, tl, :], xt_sc.at[prev, tl], sem.at[prev]).wait()

    @pl.when(k <= nc)
    def _issue():
        for tl in range(tc):
            pltpu.make_async_copy(
                x_ref.at[:, tl, :], xt_sc.at[buf, tl], sem.at[buf]).start()

    @pl.when(k == 1)
    def _init():
        h_sc[...] = jnp.zeros_like(h_sc)
        c_sc[...] = jnp.zeros_like(c_sc)

    pre_sc[prev] = (
        _dot_tb(xt_sc[prev].reshape(tc * bp, x_ref.shape[2]), wih_ref[...])
        + b_ref[...]
    )

    hs = [h_sc[g * grp:(g + 1) * grp, :] for g in range(n_grp)]
    cs = [c_sc[g * grp:(g + 1) * grp, :] for g in range(n_grp)]
    for tl in range(tc):
        row0 = tl * bp
        for g in range(n_grp):
            gates = pre_sc[buf, row0 + g * grp:row0 + (g + 1) * grp, :] + (
                jnp.dot(hs[g].astype(bf16), whht_sc[...],
                        preferred_element_type=jnp.float32))
            i_g = _sig(gates[:, 0 * h_dim:1 * h_dim])
            f_g = _sig(gates[:, 1 * h_dim:2 * h_dim])
            g_g = jnp.tanh(gates[:, 2 * h_dim:3 * h_dim])
            o_g = _sig(gates[:, 3 * h_dim:4 * h_dim])
            cs[g] = f_g * cs[g] + i_g * g_g
            hs[g] = o_g * jnp.tanh(cs[g])

    @pl.when(k >= 2)
    def _store():
        for g in range(n_grp):
            h_sc[g * grp:(g + 1) * grp, :] = hs[g]
            c_sc[g * grp:(g + 1) * grp, :] = cs[g]

    @pl.when(k == nc + 1)
    def _finalize():
        for g in range(n_grp):
            out_ref[g * grp:(g + 1) * grp, :] = (
                _dot_tb(hs[g], wfc_ref[...]) + bfc_ref[...]
            )


def kernel(x_btI, w_ih, w_hh, b_ih, b_hh, w_fc, b_fc):
    B, T, I = x_btI.shape
    H = w_hh.shape[1]
    O = w_fc.shape[0]
    f32 = jnp.float32

    grp = B // 2 if B % 16 == 0 else B

    tc = 1
    for cand in range(1, min(T, 16) + 1):
        if T % cand == 0:
            tc = cand
    nc = T // tc

    b = (b_ih + b_hh).astype(f32).reshape(1, 4 * H)
    bfc = b_fc.astype(f32).reshape(1, O)

    grid_spec = pltpu.PrefetchScalarGridSpec(
        num_scalar_prefetch=0,
        grid=(nc + 2,),
        in_specs=[
            pl.BlockSpec((B, tc, I),
                         lambda ti: (0, jnp.minimum(ti, nc - 1), 0)),
            pl.BlockSpec((4 * H, I), lambda ti: (0, 0)),
            pl.BlockSpec((4 * H, H), lambda ti: (0, 0)),
            pl.BlockSpec((1, 4 * H), lambda ti: (0, 0)),
            pl.BlockSpec((O, H), lambda ti: (0, 0)),
            pl.BlockSpec((1, O), lambda ti: (0, 0)),
        ],
        out_specs=pl.BlockSpec((B, O), lambda ti: (0, 0)),
        scratch_shapes=[
            pltpu.VMEM((B, H), f32),
            pltpu.VMEM((B, H), f32),
            pltpu.VMEM((2, tc, B, I), f32),
            pltpu.VMEM((2, tc * B, 4 * H), f32),
            pltpu.VMEM((H, 4 * H), jnp.bfloat16),
            pltpu.SemaphoreType.DMA((2,)),
        ],
    )

    out = pl.pallas_call(
        functools.partial(_lstm_kernel, tc=tc, bp=B, h_dim=H, grp=grp, nc=nc),
        out_shape=jax.ShapeDtypeStruct((B, O), f32),
        grid_spec=grid_spec,
        compiler_params=pltpu.CompilerParams(
            dimension_semantics=("arbitrary",),
            vmem_limit_bytes=60 * 1024 * 1024,
        ),
    )(x_btI.reshape(B, T, I), w_ih.astype(f32), w_hh.astype(f32), b,
      w_fc.astype(f32), bfc)

    return out


# speedup vs baseline: 1.2698x; 1.2698x over previous
import functools

import jax
import jax.numpy as jnp
from jax import lax
from jax.experimental import pallas as pl
from jax.experimental.pallas import tpu as pltpu


def _sig(v):
    return 0.5 * jnp.tanh(0.5 * v) + 0.5


def _dot_tb(a, b_raw):
    return lax.dot_general(a, b_raw, (((1,), (1,)), ((), ())),
                           preferred_element_type=jnp.float32)


def _lstm_kernel(x_ref, wih_ref, whh_ref, b_ref, wfc_ref, bfc_ref, out_ref,
                 h_sc, c_sc, xt_sc, pre_sc, whht_sc, sem,
                 *, tc, bp, h_dim, grp, nc):
    k = pl.program_id(0)
    bf16 = jnp.bfloat16
    n_grp = bp // grp
    buf = lax.rem(k, 2)
    prev = lax.rem(k + 1, 2)

    @pl.when(k == 0)
    def _init():
        h_sc[...] = jnp.zeros_like(h_sc)
        c_sc[...] = jnp.zeros_like(c_sc)
        whht_sc[...] = jnp.transpose(whh_ref[...]).astype(bf16)

    @pl.when(k < nc)
    def _issue():
        for tl in range(tc):
            pltpu.make_async_copy(
                x_ref.at[:, tl, :], xt_sc.at[buf, tl], sem.at[buf]).start()

    @pl.when(k > 0)
    def _process():
        for tl in range(tc):
            pltpu.make_async_copy(
                x_ref.at[:, tl, :], xt_sc.at[prev, tl], sem.at[prev]).wait()

        pre_sc[...] = (
            _dot_tb(xt_sc[prev].reshape(tc * bp, x_ref.shape[2]),
                    wih_ref[...])
            + b_ref[...]
        )

        hs = [h_sc[g * grp:(g + 1) * grp, :] for g in range(n_grp)]
        cs = [c_sc[g * grp:(g + 1) * grp, :] for g in range(n_grp)]
        for tl in range(tc):
            row0 = tl * bp
            for g in range(n_grp):
                gates = pre_sc[row0 + g * grp:row0 + (g + 1) * grp, :] + (
                    jnp.dot(hs[g].astype(bf16), whht_sc[...],
                            preferred_element_type=jnp.float32))
                i_g = _sig(gates[:, 0 * h_dim:1 * h_dim])
                f_g = _sig(gates[:, 1 * h_dim:2 * h_dim])
                g_g = jnp.tanh(gates[:, 2 * h_dim:3 * h_dim])
                o_g = _sig(gates[:, 3 * h_dim:4 * h_dim])
                cs[g] = f_g * cs[g] + i_g * g_g
                hs[g] = o_g * jnp.tanh(cs[g])

        for g in range(n_grp):
            h_sc[g * grp:(g + 1) * grp, :] = hs[g]
            c_sc[g * grp:(g + 1) * grp, :] = cs[g]

        @pl.when(k == nc)
        def _finalize():
            for g in range(n_grp):
                out_ref[g * grp:(g + 1) * grp, :] = (
                    _dot_tb(hs[g], wfc_ref[...]) + bfc_ref[...]
                )


def kernel(x_btI, w_ih, w_hh, b_ih, b_hh, w_fc, b_fc):
    B, T, I = x_btI.shape
    H = w_hh.shape[1]
    O = w_fc.shape[0]
    f32 = jnp.float32

    grp = B // 2 if B % 16 == 0 else B

    tc = 1
    for cand in range(1, min(T, 16) + 1):
        if T % cand == 0:
            tc = cand
    nc = T // tc

    b = (b_ih + b_hh).astype(f32).reshape(1, 4 * H)
    bfc = b_fc.astype(f32).reshape(1, O)

    grid_spec = pltpu.PrefetchScalarGridSpec(
        num_scalar_prefetch=0,
        grid=(nc + 1,),
        in_specs=[
            pl.BlockSpec((B, tc, I),
                         lambda ti: (0, jnp.minimum(ti, nc - 1), 0)),
            pl.BlockSpec((4 * H, I), lambda ti: (0, 0)),
            pl.BlockSpec((4 * H, H), lambda ti: (0, 0)),
            pl.BlockSpec((1, 4 * H), lambda ti: (0, 0)),
            pl.BlockSpec((O, H), lambda ti: (0, 0)),
            pl.BlockSpec((1, O), lambda ti: (0, 0)),
        ],
        out_specs=pl.BlockSpec((B, O), lambda ti: (0, 0)),
        scratch_shapes=[
            pltpu.VMEM((B, H), f32),
            pltpu.VMEM((B, H), f32),
            pltpu.VMEM((2, tc, B, I), f32),
            pltpu.VMEM((tc * B, 4 * H), f32),
            pltpu.VMEM((H, 4 * H), jnp.bfloat16),
            pltpu.SemaphoreType.DMA((2,)),
        ],
    )

    out = pl.pallas_call(
        functools.partial(_lstm_kernel, tc=tc, bp=B, h_dim=H, grp=grp, nc=nc),
        out_shape=jax.ShapeDtypeStruct((B, O), f32),
        grid_spec=grid_spec,
        compiler_params=pltpu.CompilerParams(
            dimension_semantics=("arbitrary",),
            vmem_limit_bytes=56 * 1024 * 1024,
        ),
    )(x_btI.reshape(B, T, I), w_ih.astype(f32), w_hh.astype(f32), b,
      w_fc.astype(f32), bfc)

    return out


# speedup vs baseline: 1.2994x; 1.0233x over previous
import functools

import jax
import jax.numpy as jnp
from jax import lax
from jax.experimental import pallas as pl
from jax.experimental.pallas import tpu as pltpu


def _sig(v):
    return 0.5 * jnp.tanh(0.5 * v) + 0.5


def _dot_tb(a, b_raw):
    return lax.dot_general(a, b_raw, (((1,), (1,)), ((), ())),
                           preferred_element_type=jnp.float32)


def _lstm_kernel(x_ref, wih_ref, whht_ref, b_ref, wfc_ref, bfc_ref, out_ref,
                 h_sc, c_sc, xt_sc, pre_sc,
                 *, tc, bp, h_dim, grp, nc):
    k = pl.program_id(0)
    bf16 = jnp.bfloat16
    n_grp = bp // grp
    buf = lax.rem(k, 2)
    prev = lax.rem(k + 1, 2)

    @pl.when(k == 0)
    def _init():
        h_sc[...] = jnp.zeros_like(h_sc)
        c_sc[...] = jnp.zeros_like(c_sc)
        for tl in range(tc):
            xt_sc[0, tl] = x_ref[:, tl, :]

    @pl.when(k > 0)
    def _process():
        pre_sc[...] = (
            _dot_tb(xt_sc[prev].reshape(tc * bp, x_ref.shape[2]),
                    wih_ref[...])
            + b_ref[...]
        )

        hs = [h_sc[g * grp:(g + 1) * grp, :] for g in range(n_grp)]
        cs = [c_sc[g * grp:(g + 1) * grp, :] for g in range(n_grp)]
        for tl in range(tc):
            xt_sc[buf, tl] = x_ref[:, tl, :]
            row0 = tl * bp
            for g in range(n_grp):
                gates = pre_sc[row0 + g * grp:row0 + (g + 1) * grp, :] + (
                    jnp.dot(hs[g].astype(bf16), whht_ref[...],
                            preferred_element_type=jnp.float32))
                i_g = _sig(gates[:, 0 * h_dim:1 * h_dim])
                f_g = _sig(gates[:, 1 * h_dim:2 * h_dim])
                g_g = jnp.tanh(gates[:, 2 * h_dim:3 * h_dim])
                o_g = _sig(gates[:, 3 * h_dim:4 * h_dim])
                cs[g] = f_g * cs[g] + i_g * g_g
                hs[g] = o_g * jnp.tanh(cs[g])

        for g in range(n_grp):
            h_sc[g * grp:(g + 1) * grp, :] = hs[g]
            c_sc[g * grp:(g + 1) * grp, :] = cs[g]

        @pl.when(k == nc)
        def _finalize():
            for g in range(n_grp):
                out_ref[g * grp:(g + 1) * grp, :] = (
                    _dot_tb(hs[g], wfc_ref[...]) + bfc_ref[...]
                )


def kernel(x_btI, w_ih, w_hh, b_ih, b_hh, w_fc, b_fc):
    B, T, I = x_btI.shape
    H = w_hh.shape[1]
    O = w_fc.shape[0]
    f32 = jnp.float32

    grp = B // 2 if B % 16 == 0 else B

    tc = 1
    for cand in range(1, min(T, 16) + 1):
        if T % cand == 0:
            tc = cand
    nc = T // tc

    whht = jnp.transpose(w_hh).astype(jnp.bfloat16)
    b = (b_ih + b_hh).astype(f32).reshape(1, 4 * H)
    bfc = b_fc.astype(f32).reshape(1, O)

    grid_spec = pltpu.PrefetchScalarGridSpec(
        num_scalar_prefetch=0,
        grid=(nc + 1,),
        in_specs=[
            pl.BlockSpec((B, tc, I),
                         lambda ti: (0, jnp.minimum(ti, nc - 1), 0)),
            pl.BlockSpec((4 * H, I), lambda ti: (0, 0)),
            pl.BlockSpec((H, 4 * H), lambda ti: (0, 0)),
            pl.BlockSpec((1, 4 * H), lambda ti: (0, 0)),
            pl.BlockSpec((O, H), lambda ti: (0, 0)),
            pl.BlockSpec((1, O), lambda ti: (0, 0)),
        ],
        out_specs=pl.BlockSpec((B, O), lambda ti: (0, 0)),
        scratch_shapes=[
            pltpu.VMEM((B, H), f32),
            pltpu.VMEM((B, H), f32),
            pltpu.VMEM((2, tc, B, I), f32),
            pltpu.VMEM((tc * B, 4 * H), f32),
        ],
    )

    out = pl.pallas_call(
        functools.partial(_lstm_kernel, tc=tc, bp=B, h_dim=H, grp=grp, nc=nc),
        out_shape=jax.ShapeDtypeStruct((B, O), f32),
        grid_spec=grid_spec,
        compiler_params=pltpu.CompilerParams(
            dimension_semantics=("arbitrary",),
            vmem_limit_bytes=56 * 1024 * 1024,
        ),
    )(x_btI.reshape(B, T, I), w_ih.astype(f32), whht, b,
      w_fc.astype(f32), bfc)

    return out


# speedup vs baseline: 1.2997x; 1.0002x over previous
import functools

import jax
import jax.numpy as jnp
from jax import lax
from jax.experimental import pallas as pl
from jax.experimental.pallas import tpu as pltpu


def _sig(v):
    return 0.5 * jnp.tanh(0.5 * v) + 0.5


def _dot_tb(a, b_raw):
    return lax.dot_general(a, b_raw, (((1,), (1,)), ((), ())),
                           preferred_element_type=jnp.float32)


def _lstm_kernel(x_ref, wih_ref, whht_ref, b_ref, wfc_ref, bfc_ref, out_ref,
                 h_sc, c_sc, xt_sc, pre_sc,
                 *, tc, bp, h_dim, grp, nc):
    k = pl.program_id(0)
    bf16 = jnp.bfloat16
    n_grp = bp // grp
    buf = lax.rem(k, 2)
    prev = lax.rem(k + 1, 2)

    @pl.when(k == 0)
    def _init():
        h_sc[...] = jnp.zeros_like(h_sc)
        c_sc[...] = jnp.zeros_like(c_sc)
        for tl in range(tc):
            xt_sc[0, tl] = x_ref[:, tl, :]

    @pl.when(k > 0)
    def _process():
        pre_sc[...] = (
            _dot_tb(xt_sc[prev].reshape(tc * bp, x_ref.shape[2]),
                    wih_ref[...])
            + b_ref[...]
        )

        hs = [h_sc[g * grp:(g + 1) * grp, :] for g in range(n_grp)]
        cs = [c_sc[g * grp:(g + 1) * grp, :] for g in range(n_grp)]
        for tl in range(tc):
            xt_sc[buf, tl] = x_ref[:, tl, :]
            row0 = tl * bp
            gates_l = [
                pre_sc[row0 + g * grp:row0 + (g + 1) * grp, :] + (
                    jnp.dot(hs[g].astype(bf16), whht_ref[...],
                            preferred_element_type=jnp.float32))
                for g in range(n_grp)
            ]
            for g in range(n_grp):
                gates = gates_l[g]
                i_g = _sig(gates[:, 0 * h_dim:1 * h_dim])
                f_g = _sig(gates[:, 1 * h_dim:2 * h_dim])
                g_g = jnp.tanh(gates[:, 2 * h_dim:3 * h_dim])
                o_g = _sig(gates[:, 3 * h_dim:4 * h_dim])
                cs[g] = f_g * cs[g] + i_g * g_g
                hs[g] = o_g * jnp.tanh(cs[g])

        for g in range(n_grp):
            h_sc[g * grp:(g + 1) * grp, :] = hs[g]
            c_sc[g * grp:(g + 1) * grp, :] = cs[g]

        @pl.when(k == nc)
        def _finalize():
            for g in range(n_grp):
                out_ref[g * grp:(g + 1) * grp, :] = (
                    _dot_tb(hs[g], wfc_ref[...]) + bfc_ref[...]
                )


def kernel(x_btI, w_ih, w_hh, b_ih, b_hh, w_fc, b_fc):
    B, T, I = x_btI.shape
    H = w_hh.shape[1]
    O = w_fc.shape[0]
    f32 = jnp.float32

    grp = B // 2 if B % 16 == 0 else B

    tc = 1
    for cand in range(1, min(T, 16) + 1):
        if T % cand == 0:
            tc = cand
    nc = T // tc

    whht = jnp.transpose(w_hh).astype(jnp.bfloat16)
    b = (b_ih + b_hh).astype(f32).reshape(1, 4 * H)
    bfc = b_fc.astype(f32).reshape(1, O)

    grid_spec = pltpu.PrefetchScalarGridSpec(
        num_scalar_prefetch=0,
        grid=(nc + 1,),
        in_specs=[
            pl.BlockSpec((B, tc, I),
                         lambda ti: (0, jnp.minimum(ti, nc - 1), 0)),
            pl.BlockSpec((4 * H, I), lambda ti: (0, 0)),
            pl.BlockSpec((H, 4 * H), lambda ti: (0, 0)),
            pl.BlockSpec((1, 4 * H), lambda ti: (0, 0)),
            pl.BlockSpec((O, H), lambda ti: (0, 0)),
            pl.BlockSpec((1, O), lambda ti: (0, 0)),
        ],
        out_specs=pl.BlockSpec((B, O), lambda ti: (0, 0)),
        scratch_shapes=[
            pltpu.VMEM((B, H), f32),
            pltpu.VMEM((B, H), f32),
            pltpu.VMEM((2, tc, B, I), f32),
            pltpu.VMEM((tc * B, 4 * H), f32),
        ],
    )

    out = pl.pallas_call(
        functools.partial(_lstm_kernel, tc=tc, bp=B, h_dim=H, grp=grp, nc=nc),
        out_shape=jax.ShapeDtypeStruct((B, O), f32),
        grid_spec=grid_spec,
        compiler_params=pltpu.CompilerParams(
            dimension_semantics=("arbitrary",),
            vmem_limit_bytes=56 * 1024 * 1024,
        ),
    )(x_btI.reshape(B, T, I), w_ih.astype(f32), whht, b,
      w_fc.astype(f32), bfc)

    return out


# speedup vs baseline: 1.5619x; 1.2017x over previous
import functools

import jax
import jax.numpy as jnp
from jax import lax
from jax.experimental import pallas as pl
from jax.experimental.pallas import tpu as pltpu


def _sig(v):
    return 0.5 * jnp.tanh(0.5 * v) + 0.5


def _dot_tb(a, b_raw):
    return lax.dot_general(a, b_raw, (((1,), (1,)), ((), ())),
                           preferred_element_type=jnp.float32)


def _lstm_kernel(x_ref, wih_ref, whht_ref, b_ref, wfc_ref, bfc_ref, out_ref,
                 h_sc, c_sc, xt_sc, pre_sc, sem,
                 *, tc, bp, h_dim, grp, nc):
    k = pl.program_id(0)
    bf16 = jnp.bfloat16
    n_grp = bp // grp
    buf = lax.rem(k, 2)
    prev = lax.rem(k + 1, 2)

    @pl.when(k == 0)
    def _init():
        h_sc[...] = jnp.zeros_like(h_sc)
        c_sc[...] = jnp.zeros_like(c_sc)

    @pl.when(k < nc)
    def _issue():
        for tl in range(tc):
            pltpu.make_async_copy(
                x_ref.at[:, k * tc + tl, :], xt_sc.at[buf, tl],
                sem.at[buf]).start()

    @pl.when(k > 0)
    def _process():
        for tl in range(tc):
            pltpu.make_async_copy(
                x_ref.at[:, tl, :], xt_sc.at[prev, tl], sem.at[prev]).wait()

        pre_sc[...] = (
            _dot_tb(xt_sc[prev].reshape(tc * bp, x_ref.shape[2]),
                    wih_ref[...])
            + b_ref[...]
        )

        hs = [h_sc[g * grp:(g + 1) * grp, :] for g in range(n_grp)]
        cs = [c_sc[g * grp:(g + 1) * grp, :] for g in range(n_grp)]
        for tl in range(tc):
            row0 = tl * bp
            gates_l = [
                pre_sc[row0 + g * grp:row0 + (g + 1) * grp, :] + (
                    jnp.dot(hs[g].astype(bf16), whht_ref[...],
                            preferred_element_type=jnp.float32))
                for g in range(n_grp)
            ]
            for g in range(n_grp):
                gates = gates_l[g]
                i_g = _sig(gates[:, 0 * h_dim:1 * h_dim])
                f_g = _sig(gates[:, 1 * h_dim:2 * h_dim])
                g_g = jnp.tanh(gates[:, 2 * h_dim:3 * h_dim])
                o_g = _sig(gates[:, 3 * h_dim:4 * h_dim])
                cs[g] = f_g * cs[g] + i_g * g_g
                hs[g] = o_g * jnp.tanh(cs[g])

        for g in range(n_grp):
            h_sc[g * grp:(g + 1) * grp, :] = hs[g]
            c_sc[g * grp:(g + 1) * grp, :] = cs[g]

        @pl.when(k == nc)
        def _finalize():
            for g in range(n_grp):
                out_ref[g * grp:(g + 1) * grp, :] = (
                    _dot_tb(hs[g], wfc_ref[...]) + bfc_ref[...]
                )


def kernel(x_btI, w_ih, w_hh, b_ih, b_hh, w_fc, b_fc):
    B, T, I = x_btI.shape
    H = w_hh.shape[1]
    O = w_fc.shape[0]
    f32 = jnp.float32

    grp = B // 2 if B % 16 == 0 else B

    tc = 1
    for cand in range(1, min(T, 16) + 1):
        if T % cand == 0:
            tc = cand
    nc = T // tc

    whht = jnp.transpose(w_hh).astype(jnp.bfloat16)
    b = (b_ih + b_hh).astype(f32).reshape(1, 4 * H)
    bfc = b_fc.astype(f32).reshape(1, O)

    grid_spec = pltpu.PrefetchScalarGridSpec(
        num_scalar_prefetch=0,
        grid=(nc + 1,),
        in_specs=[
            pl.BlockSpec(memory_space=pltpu.MemorySpace.HBM),
            pl.BlockSpec((4 * H, I), lambda ti: (0, 0)),
            pl.BlockSpec((H, 4 * H), lambda ti: (0, 0)),
            pl.BlockSpec((1, 4 * H), lambda ti: (0, 0)),
            pl.BlockSpec((O, H), lambda ti: (0, 0)),
            pl.BlockSpec((1, O), lambda ti: (0, 0)),
        ],
        out_specs=pl.BlockSpec((B, O), lambda ti: (0, 0)),
        scratch_shapes=[
            pltpu.VMEM((B, H), f32),
            pltpu.VMEM((B, H), f32),
            pltpu.VMEM((2, tc, B, I), f32),
            pltpu.VMEM((tc * B, 4 * H), f32),
            pltpu.SemaphoreType.DMA((2,)),
        ],
    )

    out = pl.pallas_call(
        functools.partial(_lstm_kernel, tc=tc, bp=B, h_dim=H, grp=grp, nc=nc),
        out_shape=jax.ShapeDtypeStruct((B, O), f32),
        grid_spec=grid_spec,
        compiler_params=pltpu.CompilerParams(
            dimension_semantics=("arbitrary",),
            vmem_limit_bytes=56 * 1024 * 1024,
        ),
    )(x_btI.reshape(B, T, I), w_ih.astype(f32), whht, b,
      w_fc.astype(f32), bfc)

    return out
